# baseline (speedup 1.0000x reference)
import numpy as np
import jax
import jax.numpy as jnp
from jax.experimental import pallas as pl
from jax.experimental.pallas import tpu as pltpu

_EDGE_SRC = np.array([0, 2, 4, 1, 3, 5, 6, 8, 10, 7, 9, 11, 13, 12, 2, 4, 14, 3, 5, 14, 8, 10, 13, 9, 11, 13, 12, 13], dtype=np.int64)
_EDGE_DST = np.array([2, 4, 14, 3, 5, 14, 8, 10, 13, 9, 11, 13, 14, 13, 0, 2, 4, 1, 3, 5, 6, 8, 10, 7, 9, 11, 13, 14], dtype=np.int64)
_V = 15


def _lhat() -> np.ndarray:
    deg = np.zeros((_V,), np.float64)
    for s in _EDGE_SRC:
        deg[s] += 1.0
    dinv = np.where(deg > 0, 1.0 / np.sqrt(np.maximum(deg, 1e-12)), 0.0)
    L = np.zeros((_V, _V), np.float64)
    for s, d in zip(_EDGE_SRC, _EDGE_DST):
        L[d, s] += -dinv[s] * dinv[d]
    return L.astype(np.float32)


_LHAT = _lhat()

_T, _B, _H, _C, _M = 300, 64, 128, 3, 2
_N = 32
_G4 = 4 * _H
_VM = _V * _M
_VMP = 128
_CH = 60
_NCH = _T // _CH


def _gate(g, c_prev):
    i = jax.nn.sigmoid(g[:, :_H])
    f = jax.nn.sigmoid(g[:, _H:2 * _H])
    gg = jnp.tanh(g[:, 2 * _H:3 * _H])
    o = jax.nn.sigmoid(g[:, 3 * _H:])
    c = f * c_prev + i * gg
    h = o * jnp.tanh(c)
    return h, c


def _lstm_kernel(xp_ref, q_ref, c0m_ref, wf_ref, c1_ref, fcw_ref, fcb_ref,
                 out_ref, gcm_ref):
    z = jnp.zeros((_B, _H), jnp.float32)

    def substep(i_glob, j, st):
        h0, c0s, h1, c1s = st
        hcat = jnp.concatenate([h0, h1], axis=1).astype(jnp.bfloat16)
        gall = jnp.dot(hcat, wf_ref[...], preferred_element_type=jnp.float32)
        g0 = gcm_ref[pl.ds(j * _N, _N), :].reshape(_B, _G4) + gall[:, :_G4]
        g1 = gall[:, _G4:] + c1_ref[...]
        h0n, c0n = _gate(g0, c0s)
        h1n, c1n = _gate(g1, c1s)
        first = i_glob == 0
        h1 = jnp.where(first, h1, h1n)
        c1s = jnp.where(first, c1s, c1n)
        return (h0n, c0n, h1, c1s)

    def chunk(ci, carry):
        base = ci * _CH
        acc = c0m_ref[...].astype(jnp.float32) * jnp.ones((_CH * _N, 1), jnp.float32)
        for c in range(_C):
            xc = xp_ref[pl.ds(base, _CH), pl.ds(c * _N, _N), :]
            acc = acc + jnp.dot(xc.reshape(_CH * _N, _VMP), q_ref[c],
                                preferred_element_type=jnp.float32)
        gcm_ref[...] = acc

        def step(jj, st):
            st = substep(ci * _CH + 2 * jj, 2 * jj, st)
            return substep(ci * _CH + 2 * jj + 1, 2 * jj + 1, st)

        return jax.lax.fori_loop(0, _CH // 2, step, carry)

    h0, c0s, h1, c1s = jax.lax.fori_loop(0, _NCH, chunk, (z, z, z, z))

    hcat = jnp.concatenate([h0, h1], axis=1).astype(jnp.bfloat16)
    gall = jnp.dot(hcat, wf_ref[...], preferred_element_type=jnp.float32)
    g1 = gall[:, _G4:] + c1_ref[...]
    h1, _ = _gate(g1, c1s)

    logits = jnp.dot(h1, fcw_ref[...],
                     preferred_element_type=jnp.float32) + fcb_ref[...]
    m = jnp.max(logits, axis=1, keepdims=True)
    e = jnp.exp(logits - m)
    out_ref[...] = e / jnp.sum(e, axis=1, keepdims=True)


def kernel(x1, x2, cheb_W0, cheb_W1, cheb_b, W_ih_l0, W_hh_l0, b_ih_l0,
           b_hh_l0, W_ih_l1, W_hh_l1, b_ih_l1, b_hh_l1, fc_W, fc_b):
    del x2
    xp = jnp.transpose(x1.reshape(_N, _C, _T, _VM), (2, 1, 0, 3))
    xp = jnp.pad(xp.astype(jnp.bfloat16),
                 ((0, 0), (0, 0), (0, 0), (0, _VMP - _VM)))
    xp = xp.reshape(_T, _C * _N, _VMP)

    lhat = jnp.asarray(_LHAT)
    eye = jnp.eye(_V, dtype=jnp.float32)
    mflat = jnp.kron(eye, cheb_W0.T) + jnp.kron(lhat.T, cheb_W1.T)
    p0 = mflat @ W_ih_l0.T

    q = jnp.zeros((_C, _VMP, _M * _G4), jnp.float32)
    v = np.arange(_V)
    for c in range(_C):
        for m in range(_M):
            q = q.at[c, v * _M + m, m * _G4:(m + 1) * _G4].set(p0[v * _C + c, :])
    q = q.astype(jnp.bfloat16)

    c0 = jnp.tile(cheb_b, _V) @ W_ih_l0.T + b_ih_l0 + b_hh_l0
    c0m = jnp.concatenate([c0, c0])[None, :]

    wf = jnp.concatenate([
        jnp.concatenate([W_hh_l0.T, W_ih_l1.T], axis=1),
        jnp.concatenate([jnp.zeros((_H, _G4), jnp.float32), W_hh_l1.T], axis=1),
    ], axis=0).astype(jnp.bfloat16)
    c1 = (b_ih_l1 + b_hh_l1)[None, :]

    return pl.pallas_call(
        _lstm_kernel,
        out_shape=jax.ShapeDtypeStruct((_B, fc_W.shape[0]), jnp.float32),
        scratch_shapes=[pltpu.VMEM((_CH * _N, _M * _G4), jnp.float32)],
    )(xp, q, c0m, wf, c1, fc_W.T, fc_b[None, :])


# speedup vs baseline: 1.1307x; 1.1307x over previous
import numpy as np
import jax
import jax.numpy as jnp
from jax.experimental import pallas as pl
from jax.experimental.pallas import tpu as pltpu

_EDGE_SRC = np.array([0, 2, 4, 1, 3, 5, 6, 8, 10, 7, 9, 11, 13, 12, 2, 4, 14, 3, 5, 14, 8, 10, 13, 9, 11, 13, 12, 13], dtype=np.int64)
_EDGE_DST = np.array([2, 4, 14, 3, 5, 14, 8, 10, 13, 9, 11, 13, 14, 13, 0, 2, 4, 1, 3, 5, 6, 8, 10, 7, 9, 11, 13, 14], dtype=np.int64)
_V = 15


def _lhat() -> np.ndarray:
    deg = np.zeros((_V,), np.float64)
    for s in _EDGE_SRC:
        deg[s] += 1.0
    dinv = np.where(deg > 0, 1.0 / np.sqrt(np.maximum(deg, 1e-12)), 0.0)
    L = np.zeros((_V, _V), np.float64)
    for s, d in zip(_EDGE_SRC, _EDGE_DST):
        L[d, s] += -dinv[s] * dinv[d]
    return L.astype(np.float32)


_LHAT = _lhat()

_T, _B, _H, _C, _M = 300, 64, 128, 3, 2
_N = 32
_G4 = 4 * _H
_VM = _V * _M
_VMP = 128
_CH = 60
_NCH = _T // _CH


def _gate(g, c_prev):
    i = jax.nn.sigmoid(g[:, :_H])
    f = jax.nn.sigmoid(g[:, _H:2 * _H])
    gg = jnp.tanh(g[:, 2 * _H:3 * _H])
    o = jax.nn.sigmoid(g[:, 3 * _H:])
    c = f * c_prev + i * gg
    h = o * jnp.tanh(c)
    return h, c


def _lstm_kernel(xp_ref, q_ref, c0m_ref, wf_ref, c1_ref, fcw_ref, fcb_ref,
                 out_ref, gcm_ref):
    z = jnp.zeros((_B, _H), jnp.float32)

    def substep(i_glob, j, st):
        h0, c0s, h1, c1s = st
        hcat = jnp.concatenate([h0, h1], axis=1).astype(jnp.bfloat16)
        gall = jnp.dot(hcat, wf_ref[...], preferred_element_type=jnp.float32)
        g0 = (gcm_ref[pl.ds(j * _N, _N), :].reshape(_B, _G4)
              + gall[:, :_G4] + c0m_ref[...])
        g1 = gall[:, _G4:] + c1_ref[...]
        h0n, c0n = _gate(g0, c0s)
        h1n, c1n = _gate(g1, c1s)
        first = i_glob == 0
        h1 = jnp.where(first, h1, h1n)
        c1s = jnp.where(first, c1s, c1n)
        return (h0n, c0n, h1, c1s)

    def chunk(ci, carry):
        base = ci * _CH
        xc = xp_ref[pl.ds(base, _CH), :, :].reshape(_CH * _N, _VMP)
        gcm_ref[...] = jnp.dot(xc, q_ref[...],
                               preferred_element_type=jnp.float32)

        def step(jj, st):
            st = substep(ci * _CH + 2 * jj, 2 * jj, st)
            return substep(ci * _CH + 2 * jj + 1, 2 * jj + 1, st)

        return jax.lax.fori_loop(0, _CH // 2, step, carry)

    h0, c0s, h1, c1s = jax.lax.fori_loop(0, _NCH, chunk, (z, z, z, z))

    hcat = jnp.concatenate([h0, h1], axis=1).astype(jnp.bfloat16)
    gall = jnp.dot(hcat, wf_ref[...], preferred_element_type=jnp.float32)
    g1 = gall[:, _G4:] + c1_ref[...]
    h1, _ = _gate(g1, c1s)

    logits = jnp.dot(h1, fcw_ref[...],
                     preferred_element_type=jnp.float32) + fcb_ref[...]
    m = jnp.max(logits, axis=1, keepdims=True)
    e = jnp.exp(logits - m)
    out_ref[...] = e / jnp.sum(e, axis=1, keepdims=True)


def kernel(x1, x2, cheb_W0, cheb_W1, cheb_b, W_ih_l0, W_hh_l0, b_ih_l0,
           b_hh_l0, W_ih_l1, W_hh_l1, b_ih_l1, b_hh_l1, fc_W, fc_b):
    del x2
    xp = jnp.transpose(x1.reshape(_N, _C, _T, _VM), (2, 0, 1, 3))
    xp = xp.reshape(_T, _N, _C * _VM)
    xp = jnp.pad(xp.astype(jnp.bfloat16),
                 ((0, 0), (0, 0), (0, _VMP - _C * _VM)))

    lhat = jnp.asarray(_LHAT)
    eye = jnp.eye(_V, dtype=jnp.float32)
    mflat = jnp.kron(eye, cheb_W0.T) + jnp.kron(lhat.T, cheb_W1.T)
    p0 = mflat @ W_ih_l0.T

    q = jnp.zeros((_VMP, _M * _G4), jnp.float32)
    v = np.arange(_V)
    for c in range(_C):
        for m in range(_M):
            q = q.at[c * _VM + v * _M + m, m * _G4:(m + 1) * _G4].set(
                p0[v * _C + c, :])
    q = q.astype(jnp.bfloat16)

    c0m = (jnp.tile(cheb_b, _V) @ W_ih_l0.T + b_ih_l0 + b_hh_l0)[None, :]

    wf = jnp.concatenate([
        jnp.concatenate([W_hh_l0.T, W_ih_l1.T], axis=1),
        jnp.concatenate([jnp.zeros((_H, _G4), jnp.float32), W_hh_l1.T], axis=1),
    ], axis=0).astype(jnp.bfloat16)
    c1 = (b_ih_l1 + b_hh_l1)[None, :]

    return pl.pallas_call(
        _lstm_kernel,
        out_shape=jax.ShapeDtypeStruct((_B, fc_W.shape[0]), jnp.float32),
        scratch_shapes=[pltpu.VMEM((_CH * _N, _M * _G4), jnp.float32)],
    )(xp, q, c0m, wf, c1, fc_W.T, fc_b[None, :])


# speedup vs baseline: 7.3914x; 6.5369x over previous
import numpy as np
import jax
import jax.numpy as jnp
from jax.experimental import pallas as pl
from jax.experimental.pallas import tpu as pltpu

_EDGE_SRC = np.array([0, 2, 4, 1, 3, 5, 6, 8, 10, 7, 9, 11, 13, 12, 2, 4, 14, 3, 5, 14, 8, 10, 13, 9, 11, 13, 12, 13], dtype=np.int64)
_EDGE_DST = np.array([2, 4, 14, 3, 5, 14, 8, 10, 13, 9, 11, 13, 14, 13, 0, 2, 4, 1, 3, 5, 6, 8, 10, 7, 9, 11, 13, 14], dtype=np.int64)
_V = 15


def _lhat() -> np.ndarray:
    deg = np.zeros((_V,), np.float64)
    for s in _EDGE_SRC:
        deg[s] += 1.0
    dinv = np.where(deg > 0, 1.0 / np.sqrt(np.maximum(deg, 1e-12)), 0.0)
    L = np.zeros((_V, _V), np.float64)
    for s, d in zip(_EDGE_SRC, _EDGE_DST):
        L[d, s] += -dinv[s] * dinv[d]
    return L.astype(np.float32)


_LHAT = _lhat()

_T, _B, _H, _C, _M = 300, 64, 128, 3, 2
_N = 32
_G4 = 4 * _H
_VM = _V * _M
_VMP = 128
_CH = 60
_NCH = _T // _CH
_UNPERM = np.array([(b % 2) * 32 + b // 2 for b in range(64)], dtype=np.int32)


def _gate(g, c_prev):
    i = jax.nn.sigmoid(g[:, :_H])
    f = jax.nn.sigmoid(g[:, _H:2 * _H])
    gg = jnp.tanh(g[:, 2 * _H:3 * _H])
    o = jax.nn.sigmoid(g[:, 3 * _H:])
    c = f * c_prev + i * gg
    h = o * jnp.tanh(c)
    return h, c


def _lstm_kernel(xp_ref, q_ref, c0m_ref, wf_ref, c1_ref, fcw_ref, fcb_ref,
                 out_ref, gcm_ref):
    z = jnp.zeros((_B, _H), jnp.float32)

    def substep(i_glob, j, st):
        h0, c0s, h1, c1s = st
        hcat = jnp.concatenate([h0, h1], axis=1).astype(jnp.bfloat16)
        gall = jnp.dot(hcat, wf_ref[...], preferred_element_type=jnp.float32)
        a0 = gcm_ref[0, pl.ds(j * _N, _N), :]
        a1 = gcm_ref[1, pl.ds(j * _N, _N), :]
        g0 = jnp.concatenate([a0, a1], axis=0) + gall[:, :_G4] + c0m_ref[...]
        g1 = gall[:, _G4:] + c1_ref[...]
        h0n, c0n = _gate(g0, c0s)
        h1n, c1n = _gate(g1, c1s)
        first = i_glob == 0
        h1 = jnp.where(first, h1, h1n)
        c1s = jnp.where(first, c1s, c1n)
        return (h0n, c0n, h1, c1s)

    def chunk(ci, carry):
        base = ci * _CH
        xc = xp_ref[pl.ds(base, _CH), :, :].reshape(_CH * _N, _VMP)
        gcm_ref[0] = jnp.dot(xc, q_ref[0], preferred_element_type=jnp.float32)
        gcm_ref[1] = jnp.dot(xc, q_ref[1], preferred_element_type=jnp.float32)

        def step(jj, st):
            st = substep(ci * _CH + 2 * jj, 2 * jj, st)
            return substep(ci * _CH + 2 * jj + 1, 2 * jj + 1, st)

        return jax.lax.fori_loop(0, _CH // 2, step, carry)

    h0, c0s, h1, c1s = jax.lax.fori_loop(0, _NCH, chunk, (z, z, z, z))

    hcat = jnp.concatenate([h0, h1], axis=1).astype(jnp.bfloat16)
    gall = jnp.dot(hcat, wf_ref[...], preferred_element_type=jnp.float32)
    g1 = gall[:, _G4:] + c1_ref[...]
    h1, _ = _gate(g1, c1s)

    logits = jnp.dot(h1, fcw_ref[...],
                     preferred_element_type=jnp.float32) + fcb_ref[...]
    m = jnp.max(logits, axis=1, keepdims=True)
    e = jnp.exp(logits - m)
    sm = e / jnp.sum(e, axis=1, keepdims=True)
    out_ref[...] = jnp.transpose(sm.reshape(_M, _N, -1), (1, 0, 2)).reshape(_B, -1)


def kernel(x1, x2, cheb_W0, cheb_W1, cheb_b, W_ih_l0, W_hh_l0, b_ih_l0,
           b_hh_l0, W_ih_l1, W_hh_l1, b_ih_l1, b_hh_l1, fc_W, fc_b):
    del x2
    xp = jnp.transpose(x1.reshape(_N, _C, _T, _VM), (2, 0, 1, 3))
    xp = xp.reshape(_T, _N, _C * _VM)
    xp = jnp.pad(xp.astype(jnp.bfloat16),
                 ((0, 0), (0, 0), (0, _VMP - _C * _VM)))

    lhat = jnp.asarray(_LHAT)
    eye = jnp.eye(_V, dtype=jnp.float32)
    mflat = jnp.kron(eye, cheb_W0.T) + jnp.kron(lhat.T, cheb_W1.T)
    p0 = mflat @ W_ih_l0.T

    q = jnp.zeros((_M, _VMP, _G4), jnp.float32)
    v = np.arange(_V)
    for c in range(_C):
        for m in range(_M):
            q = q.at[m, c * _VM + v * _M + m, :].set(p0[v * _C + c, :])
    q = q.astype(jnp.bfloat16)

    c0m = (jnp.tile(cheb_b, _V) @ W_ih_l0.T + b_ih_l0 + b_hh_l0)[None, :]

    wf = jnp.concatenate([
        jnp.concatenate([W_hh_l0.T, W_ih_l1.T], axis=1),
        jnp.concatenate([jnp.zeros((_H, _G4), jnp.float32), W_hh_l1.T], axis=1),
    ], axis=0).astype(jnp.bfloat16)
    c1 = (b_ih_l1 + b_hh_l1)[None, :]

    return pl.pallas_call(
        _lstm_kernel,
        out_shape=jax.ShapeDtypeStruct((_B, fc_W.shape[0]), jnp.float32),
        scratch_shapes=[pltpu.VMEM((_M, _CH * _N, _G4), jnp.float32)],
    )(xp, q, c0m, wf, c1, fc_W.T, fc_b[None, :])


# speedup vs baseline: 7.7508x; 1.0486x over previous
import numpy as np
import jax
import jax.numpy as jnp
from jax.experimental import pallas as pl
from jax.experimental.pallas import tpu as pltpu

_EDGE_SRC = np.array([0, 2, 4, 1, 3, 5, 6, 8, 10, 7, 9, 11, 13, 12, 2, 4, 14, 3, 5, 14, 8, 10, 13, 9, 11, 13, 12, 13], dtype=np.int64)
_EDGE_DST = np.array([2, 4, 14, 3, 5, 14, 8, 10, 13, 9, 11, 13, 14, 13, 0, 2, 4, 1, 3, 5, 6, 8, 10, 7, 9, 11, 13, 14], dtype=np.int64)
_V = 15


def _lhat() -> np.ndarray:
    deg = np.zeros((_V,), np.float64)
    for s in _EDGE_SRC:
        deg[s] += 1.0
    dinv = np.where(deg > 0, 1.0 / np.sqrt(np.maximum(deg, 1e-12)), 0.0)
    L = np.zeros((_V, _V), np.float64)
    for s, d in zip(_EDGE_SRC, _EDGE_DST):
        L[d, s] += -dinv[s] * dinv[d]
    return L.astype(np.float32)


_LHAT = _lhat()

_T, _B, _H, _C, _M = 300, 64, 128, 3, 2
_N = 32
_G4 = 4 * _H
_VM = _V * _M
_VMP = 128
_CH = 60
_NCH = _T // _CH
_UNPERM = np.array([(b % 2) * 32 + b // 2 for b in range(64)], dtype=np.int32)


def _gate(g, c_prev):
    i = jax.nn.sigmoid(g[:, :_H])
    f = jax.nn.sigmoid(g[:, _H:2 * _H])
    gg = jnp.tanh(g[:, 2 * _H:3 * _H])
    o = jax.nn.sigmoid(g[:, 3 * _H:])
    c = f * c_prev + i * gg
    h = o * jnp.tanh(c)
    return h, c


def _lstm_kernel(xp_ref, q_ref, c0m_ref, wf_ref, c1_ref, fcw_ref, fcb_ref,
                 out_ref, gcm_ref):
    z = jnp.zeros((_B, _H), jnp.float32)

    def substep(i_glob, j, st):
        h0, c0s, h1, c1s = st
        hcat = jnp.concatenate([h0, h1], axis=1).astype(jnp.bfloat16)
        gall = jnp.dot(hcat, wf_ref[...], preferred_element_type=jnp.float32)
        a0 = gcm_ref[0, pl.ds(j * _N, _N), :]
        a1 = gcm_ref[1, pl.ds(j * _N, _N), :]
        g0 = jnp.concatenate([a0, a1], axis=0) + gall[:, :_G4] + c0m_ref[...]
        g1 = gall[:, _G4:] + c1_ref[...]
        h0n, c0n = _gate(g0, c0s)
        h1n, c1n = _gate(g1, c1s)
        first = i_glob == 0
        h1 = jnp.where(first, h1, h1n)
        c1s = jnp.where(first, c1s, c1n)
        return (h0n, c0n, h1, c1s)

    def chunk(ci, carry):
        base = ci * _CH
        xc = xp_ref[pl.ds(base, _CH), :, :].reshape(_CH * _N, _VMP)
        gcm_ref[0] = jnp.dot(xc, q_ref[0], preferred_element_type=jnp.float32)
        gcm_ref[1] = jnp.dot(xc, q_ref[1], preferred_element_type=jnp.float32)

        def step(jj, st):
            for u in range(4):
                st = substep(ci * _CH + 4 * jj + u, 4 * jj + u, st)
            return st

        return jax.lax.fori_loop(0, _CH // 4, step, carry)

    h0, c0s, h1, c1s = jax.lax.fori_loop(0, _NCH, chunk, (z, z, z, z))

    hcat = jnp.concatenate([h0, h1], axis=1).astype(jnp.bfloat16)
    gall = jnp.dot(hcat, wf_ref[...], preferred_element_type=jnp.float32)
    g1 = gall[:, _G4:] + c1_ref[...]
    h1, _ = _gate(g1, c1s)

    logits = jnp.dot(h1, fcw_ref[...],
                     preferred_element_type=jnp.float32) + fcb_ref[...]
    m = jnp.max(logits, axis=1, keepdims=True)
    e = jnp.exp(logits - m)
    sm = e / jnp.sum(e, axis=1, keepdims=True)
    out_ref[...] = jnp.transpose(sm.reshape(_M, _N, -1), (1, 0, 2)).reshape(_B, -1)


def kernel(x1, x2, cheb_W0, cheb_W1, cheb_b, W_ih_l0, W_hh_l0, b_ih_l0,
           b_hh_l0, W_ih_l1, W_hh_l1, b_ih_l1, b_hh_l1, fc_W, fc_b):
    del x2
    xp = jnp.transpose(x1.reshape(_N, _C, _T, _VM), (2, 0, 1, 3))
    xp = xp.reshape(_T, _N, _C * _VM)
    xp = jnp.pad(xp.astype(jnp.bfloat16),
                 ((0, 0), (0, 0), (0, _VMP - _C * _VM)))

    lhat = jnp.asarray(_LHAT)
    eye = jnp.eye(_V, dtype=jnp.float32)
    mflat = jnp.kron(eye, cheb_W0.T) + jnp.kron(lhat.T, cheb_W1.T)
    p0 = mflat @ W_ih_l0.T

    q = jnp.zeros((_M, _VMP, _G4), jnp.float32)
    v = np.arange(_V)
    for c in range(_C):
        for m in range(_M):
            q = q.at[m, c * _VM + v * _M + m, :].set(p0[v * _C + c, :])
    q = q.astype(jnp.bfloat16)

    c0m = (jnp.tile(cheb_b, _V) @ W_ih_l0.T + b_ih_l0 + b_hh_l0)[None, :]

    wf = jnp.concatenate([
        jnp.concatenate([W_hh_l0.T, W_ih_l1.T], axis=1),
        jnp.concatenate([jnp.zeros((_H, _G4), jnp.float32), W_hh_l1.T], axis=1),
    ], axis=0).astype(jnp.bfloat16)
    c1 = (b_ih_l1 + b_hh_l1)[None, :]

    return pl.pallas_call(
        _lstm_kernel,
        out_shape=jax.ShapeDtypeStruct((_B, fc_W.shape[0]), jnp.float32),
        scratch_shapes=[pltpu.VMEM((_M, _CH * _N, _G4), jnp.float32)],
    )(xp, q, c0m, wf, c1, fc_W.T, fc_b[None, :])


# speedup vs baseline: 7.9242x; 1.0224x over previous
import numpy as np
import jax
import jax.numpy as jnp
from jax.experimental import pallas as pl
from jax.experimental.pallas import tpu as pltpu

_EDGE_SRC = np.array([0, 2, 4, 1, 3, 5, 6, 8, 10, 7, 9, 11, 13, 12, 2, 4, 14, 3, 5, 14, 8, 10, 13, 9, 11, 13, 12, 13], dtype=np.int64)
_EDGE_DST = np.array([2, 4, 14, 3, 5, 14, 8, 10, 13, 9, 11, 13, 14, 13, 0, 2, 4, 1, 3, 5, 6, 8, 10, 7, 9, 11, 13, 14], dtype=np.int64)
_V = 15


def _lhat() -> np.ndarray:
    deg = np.zeros((_V,), np.float64)
    for s in _EDGE_SRC:
        deg[s] += 1.0
    dinv = np.where(deg > 0, 1.0 / np.sqrt(np.maximum(deg, 1e-12)), 0.0)
    L = np.zeros((_V, _V), np.float64)
    for s, d in zip(_EDGE_SRC, _EDGE_DST):
        L[d, s] += -dinv[s] * dinv[d]
    return L.astype(np.float32)


_LHAT = _lhat()

_T, _B, _H, _C, _M = 300, 64, 128, 3, 2
_N = 32
_G4 = 4 * _H
_VM = _V * _M
_VMP = 128
_CH = 60
_NCH = _T // _CH
_UNPERM = np.array([(b % 2) * 32 + b // 2 for b in range(64)], dtype=np.int32)


def _gate(g, c_prev):
    i = jax.nn.sigmoid(g[:, :_H])
    f = jax.nn.sigmoid(g[:, _H:2 * _H])
    gg = jnp.tanh(g[:, 2 * _H:3 * _H])
    o = jax.nn.sigmoid(g[:, 3 * _H:])
    c = f * c_prev + i * gg
    h = o * jnp.tanh(c)
    return h, c


def _lstm_kernel(xp_ref, q_ref, c0m_ref, wf_ref, c1_ref, fcw_ref, fcb_ref,
                 out_ref, gcm_ref):
    z = jnp.zeros((_B, _H), jnp.float32)

    def substep(i_glob, j, st):
        h0, c0s, h1, c1s = st
        hcat = jnp.concatenate([h0, h1], axis=1).astype(jnp.bfloat16)
        gall = jnp.dot(hcat, wf_ref[...], preferred_element_type=jnp.float32)
        a0 = gcm_ref[0, pl.ds(j * _N, _N), :]
        a1 = gcm_ref[1, pl.ds(j * _N, _N), :]
        g0 = jnp.concatenate([a0, a1], axis=0) + gall[:, :_G4] + c0m_ref[...]
        g1 = gall[:, _G4:] + c1_ref[...]
        h0n, c0n = _gate(g0, c0s)
        h1n, c1n = _gate(g1, c1s)
        first = i_glob == 0
        h1 = jnp.where(first, h1, h1n)
        c1s = jnp.where(first, c1s, c1n)
        return (h0n, c0n, h1, c1s)

    def chunk(ci, carry):
        base = ci * _CH
        xc = xp_ref[pl.ds(base, _CH), :, :].reshape(_CH * _N, _VMP)
        gcm_ref[0] = jnp.dot(xc, q_ref[0], preferred_element_type=jnp.float32)
        gcm_ref[1] = jnp.dot(xc, q_ref[1], preferred_element_type=jnp.float32)

        def step(jj, st):
            for u in range(10):
                st = substep(ci * _CH + 10 * jj + u, 10 * jj + u, st)
            return st

        return jax.lax.fori_loop(0, _CH // 10, step, carry)

    h0, c0s, h1, c1s = jax.lax.fori_loop(0, _NCH, chunk, (z, z, z, z))

    hcat = jnp.concatenate([h0, h1], axis=1).astype(jnp.bfloat16)
    gall = jnp.dot(hcat, wf_ref[...], preferred_element_type=jnp.float32)
    g1 = gall[:, _G4:] + c1_ref[...]
    h1, _ = _gate(g1, c1s)

    logits = jnp.dot(h1, fcw_ref[...],
                     preferred_element_type=jnp.float32) + fcb_ref[...]
    m = jnp.max(logits, axis=1, keepdims=True)
    e = jnp.exp(logits - m)
    sm = e / jnp.sum(e, axis=1, keepdims=True)
    out_ref[...] = jnp.transpose(sm.reshape(_M, _N, -1), (1, 0, 2)).reshape(_B, -1)


def kernel(x1, x2, cheb_W0, cheb_W1, cheb_b, W_ih_l0, W_hh_l0, b_ih_l0,
           b_hh_l0, W_ih_l1, W_hh_l1, b_ih_l1, b_hh_l1, fc_W, fc_b):
    del x2
    xp = jnp.transpose(x1.reshape(_N, _C, _T, _VM), (2, 0, 1, 3))
    xp = xp.reshape(_T, _N, _C * _VM)
    xp = jnp.pad(xp.astype(jnp.bfloat16),
                 ((0, 0), (0, 0), (0, _VMP - _C * _VM)))

    lhat = jnp.asarray(_LHAT)
    eye = jnp.eye(_V, dtype=jnp.float32)
    mflat = jnp.kron(eye, cheb_W0.T) + jnp.kron(lhat.T, cheb_W1.T)
    p0 = mflat @ W_ih_l0.T

    q = jnp.zeros((_M, _VMP, _G4), jnp.float32)
    v = np.arange(_V)
    for c in range(_C):
        for m in range(_M):
            q = q.at[m, c * _VM + v * _M + m, :].set(p0[v * _C + c, :])
    q = q.astype(jnp.bfloat16)

    c0m = (jnp.tile(cheb_b, _V) @ W_ih_l0.T + b_ih_l0 + b_hh_l0)[None, :]

    wf = jnp.concatenate([
        jnp.concatenate([W_hh_l0.T, W_ih_l1.T], axis=1),
        jnp.concatenate([jnp.zeros((_H, _G4), jnp.float32), W_hh_l1.T], axis=1),
    ], axis=0).astype(jnp.bfloat16)
    c1 = (b_ih_l1 + b_hh_l1)[None, :]

    return pl.pallas_call(
        _lstm_kernel,
        out_shape=jax.ShapeDtypeStruct((_B, fc_W.shape[0]), jnp.float32),
        scratch_shapes=[pltpu.VMEM((_M, _CH * _N, _G4), jnp.float32)],
    )(xp, q, c0m, wf, c1, fc_W.T, fc_b[None, :])


# speedup vs baseline: 8.6615x; 1.0930x over previous
import numpy as np
import jax
import jax.numpy as jnp
from jax.experimental import pallas as pl
from jax.experimental.pallas import tpu as pltpu

_EDGE_SRC = np.array([0, 2, 4, 1, 3, 5, 6, 8, 10, 7, 9, 11, 13, 12, 2, 4, 14, 3, 5, 14, 8, 10, 13, 9, 11, 13, 12, 13], dtype=np.int64)
_EDGE_DST = np.array([2, 4, 14, 3, 5, 14, 8, 10, 13, 9, 11, 13, 14, 13, 0, 2, 4, 1, 3, 5, 6, 8, 10, 7, 9, 11, 13, 14], dtype=np.int64)
_V = 15


def _lhat() -> np.ndarray:
    deg = np.zeros((_V,), np.float64)
    for s in _EDGE_SRC:
        deg[s] += 1.0
    dinv = np.where(deg > 0, 1.0 / np.sqrt(np.maximum(deg, 1e-12)), 0.0)
    L = np.zeros((_V, _V), np.float64)
    for s, d in zip(_EDGE_SRC, _EDGE_DST):
        L[d, s] += -dinv[s] * dinv[d]
    return L.astype(np.float32)


_LHAT = _lhat()

_T, _B, _H, _C, _M = 300, 64, 128, 3, 2
_N = 32
_G4 = 4 * _H
_VM = _V * _M
_VMP = 128
_CH = 60
_NCH = _T // _CH
_UNPERM = np.array([(b % 2) * 32 + b // 2 for b in range(64)], dtype=np.int32)


def _gate(g, c_prev):
    i = jax.nn.sigmoid(g[:, :_H])
    f = jax.nn.sigmoid(g[:, _H:2 * _H])
    gg = jnp.tanh(g[:, 2 * _H:3 * _H])
    o = jax.nn.sigmoid(g[:, 3 * _H:])
    c = f * c_prev + i * gg
    h = o * jnp.tanh(c)
    return h, c


def _lstm_kernel(xp_ref, q_ref, c0m_ref, wf_ref, w11_ref, c1_ref, fcw_ref,
                 fcb_ref, out_ref, gcm_ref):
    z = jnp.zeros((_B, _H), jnp.float32)

    def substep(i_glob, j, st):
        h0, c0s, h1, c1s = st
        ga = jnp.dot(h0.astype(jnp.bfloat16), wf_ref[...],
                     preferred_element_type=jnp.float32)
        gb = jnp.dot(h1.astype(jnp.bfloat16), w11_ref[...],
                     preferred_element_type=jnp.float32)
        a0 = gcm_ref[0, pl.ds(j * _N, _N), :]
        a1 = gcm_ref[1, pl.ds(j * _N, _N), :]
        g0 = jnp.concatenate([a0, a1], axis=0) + ga[:, :_G4] + c0m_ref[...]
        g1 = ga[:, _G4:] + gb + c1_ref[...]
        h0n, c0n = _gate(g0, c0s)
        h1n, c1n = _gate(g1, c1s)
        first = i_glob == 0
        h1 = jnp.where(first, h1, h1n)
        c1s = jnp.where(first, c1s, c1n)
        return (h0n, c0n, h1, c1s)

    def chunk(ci, carry):
        base = ci * _CH
        xc = xp_ref[pl.ds(base, _CH), :, :].reshape(_CH * _N, _VMP)
        gcm_ref[0] = jnp.dot(xc, q_ref[0], preferred_element_type=jnp.float32)
        gcm_ref[1] = jnp.dot(xc, q_ref[1], preferred_element_type=jnp.float32)

        def step(jj, st):
            for u in range(10):
                st = substep(ci * _CH + 10 * jj + u, 10 * jj + u, st)
            return st

        return jax.lax.fori_loop(0, _CH // 10, step, carry)

    h0, c0s, h1, c1s = jax.lax.fori_loop(0, _NCH, chunk, (z, z, z, z))

    ga = jnp.dot(h0.astype(jnp.bfloat16), wf_ref[...],
                 preferred_element_type=jnp.float32)
    gb = jnp.dot(h1.astype(jnp.bfloat16), w11_ref[...],
                 preferred_element_type=jnp.float32)
    g1 = ga[:, _G4:] + gb + c1_ref[...]
    h1, _ = _gate(g1, c1s)

    logits = jnp.dot(h1, fcw_ref[...],
                     preferred_element_type=jnp.float32) + fcb_ref[...]
    m = jnp.max(logits, axis=1, keepdims=True)
    e = jnp.exp(logits - m)
    sm = e / jnp.sum(e, axis=1, keepdims=True)
    out_ref[...] = jnp.transpose(sm.reshape(_M, _N, -1), (1, 0, 2)).reshape(_B, -1)


def kernel(x1, x2, cheb_W0, cheb_W1, cheb_b, W_ih_l0, W_hh_l0, b_ih_l0,
           b_hh_l0, W_ih_l1, W_hh_l1, b_ih_l1, b_hh_l1, fc_W, fc_b):
    del x2
    xp = jnp.transpose(x1.reshape(_N, _C, _T, _VM), (2, 0, 1, 3))
    xp = xp.reshape(_T, _N, _C * _VM)
    xp = jnp.pad(xp.astype(jnp.bfloat16),
                 ((0, 0), (0, 0), (0, _VMP - _C * _VM)))

    lhat = jnp.asarray(_LHAT)
    eye = jnp.eye(_V, dtype=jnp.float32)
    mflat = jnp.kron(eye, cheb_W0.T) + jnp.kron(lhat.T, cheb_W1.T)
    p0 = mflat @ W_ih_l0.T

    q = jnp.zeros((_M, _VMP, _G4), jnp.float32)
    v = np.arange(_V)
    for c in range(_C):
        for m in range(_M):
            q = q.at[m, c * _VM + v * _M + m, :].set(p0[v * _C + c, :])
    q = q.astype(jnp.bfloat16)

    c0m = (jnp.tile(cheb_b, _V) @ W_ih_l0.T + b_ih_l0 + b_hh_l0)[None, :]

    wf = jnp.concatenate([W_hh_l0.T, W_ih_l1.T],
                         axis=1).astype(jnp.bfloat16)
    w11 = W_hh_l1.T.astype(jnp.bfloat16)
    c1 = (b_ih_l1 + b_hh_l1)[None, :]

    return pl.pallas_call(
        _lstm_kernel,
        out_shape=jax.ShapeDtypeStruct((_B, fc_W.shape[0]), jnp.float32),
        scratch_shapes=[pltpu.VMEM((_M, _CH * _N, _G4), jnp.float32)],
    )(xp, q, c0m, wf, w11, c1, fc_W.T, fc_b[None, :])


# speedup vs baseline: 8.7205x; 1.0068x over previous
import numpy as np
import jax
import jax.numpy as jnp
from jax.experimental import pallas as pl
from jax.experimental.pallas import tpu as pltpu

_EDGE_SRC = np.array([0, 2, 4, 1, 3, 5, 6, 8, 10, 7, 9, 11, 13, 12, 2, 4, 14, 3, 5, 14, 8, 10, 13, 9, 11, 13, 12, 13], dtype=np.int64)
_EDGE_DST = np.array([2, 4, 14, 3, 5, 14, 8, 10, 13, 9, 11, 13, 14, 13, 0, 2, 4, 1, 3, 5, 6, 8, 10, 7, 9, 11, 13, 14], dtype=np.int64)
_V = 15


def _lhat() -> np.ndarray:
    deg = np.zeros((_V,), np.float64)
    for s in _EDGE_SRC:
        deg[s] += 1.0
    dinv = np.where(deg > 0, 1.0 / np.sqrt(np.maximum(deg, 1e-12)), 0.0)
    L = np.zeros((_V, _V), np.float64)
    for s, d in zip(_EDGE_SRC, _EDGE_DST):
        L[d, s] += -dinv[s] * dinv[d]
    return L.astype(np.float32)


_LHAT = _lhat()

_T, _B, _H, _C, _M = 300, 64, 128, 3, 2
_N = 32
_G4 = 4 * _H
_VM = _V * _M
_VMP = 128
_CH = 150
_NCH = _T // _CH
_UNPERM = np.array([(b % 2) * 32 + b // 2 for b in range(64)], dtype=np.int32)


def _gate(g, c_prev):
    i = jax.nn.sigmoid(g[:, :_H])
    f = jax.nn.sigmoid(g[:, _H:2 * _H])
    gg = jnp.tanh(g[:, 2 * _H:3 * _H])
    o = jax.nn.sigmoid(g[:, 3 * _H:])
    c = f * c_prev + i * gg
    h = o * jnp.tanh(c)
    return h, c


def _lstm_kernel(xp_ref, q_ref, c0m_ref, wf_ref, w11_ref, c1_ref, fcw_ref,
                 fcb_ref, out_ref, gcm_ref):
    z = jnp.zeros((_B, _H), jnp.float32)

    def substep(i_glob, j, st):
        h0, c0s, h1, c1s = st
        ga = jnp.dot(h0.astype(jnp.bfloat16), wf_ref[...],
                     preferred_element_type=jnp.float32)
        gb = jnp.dot(h1.astype(jnp.bfloat16), w11_ref[...],
                     preferred_element_type=jnp.float32)
        a0 = gcm_ref[0, pl.ds(j * _N, _N), :]
        a1 = gcm_ref[1, pl.ds(j * _N, _N), :]
        g0 = jnp.concatenate([a0, a1], axis=0) + ga[:, :_G4] + c0m_ref[...]
        g1 = ga[:, _G4:] + gb + c1_ref[...]
        h0n, c0n = _gate(g0, c0s)
        h1n, c1n = _gate(g1, c1s)
        first = i_glob == 0
        h1 = jnp.where(first, h1, h1n)
        c1s = jnp.where(first, c1s, c1n)
        return (h0n, c0n, h1, c1s)

    def chunk(ci, carry):
        base = ci * _CH
        xc = xp_ref[pl.ds(base, _CH), :, :].reshape(_CH * _N, _VMP)
        gcm_ref[0] = jnp.dot(xc, q_ref[0], preferred_element_type=jnp.float32)
        gcm_ref[1] = jnp.dot(xc, q_ref[1], preferred_element_type=jnp.float32)

        def step(jj, st):
            for u in range(10):
                st = substep(ci * _CH + 10 * jj + u, 10 * jj + u, st)
            return st

        return jax.lax.fori_loop(0, _CH // 10, step, carry)

    h0, c0s, h1, c1s = jax.lax.fori_loop(0, _NCH, chunk, (z, z, z, z))

    ga = jnp.dot(h0.astype(jnp.bfloat16), wf_ref[...],
                 preferred_element_type=jnp.float32)
    gb = jnp.dot(h1.astype(jnp.bfloat16), w11_ref[...],
                 preferred_element_type=jnp.float32)
    g1 = ga[:, _G4:] + gb + c1_ref[...]
    h1, _ = _gate(g1, c1s)

    logits = jnp.dot(h1, fcw_ref[...],
                     preferred_element_type=jnp.float32) + fcb_ref[...]
    m = jnp.max(logits, axis=1, keepdims=True)
    e = jnp.exp(logits - m)
    sm = e / jnp.sum(e, axis=1, keepdims=True)
    out_ref[...] = jnp.transpose(sm.reshape(_M, _N, -1), (1, 0, 2)).reshape(_B, -1)


def kernel(x1, x2, cheb_W0, cheb_W1, cheb_b, W_ih_l0, W_hh_l0, b_ih_l0,
           b_hh_l0, W_ih_l1, W_hh_l1, b_ih_l1, b_hh_l1, fc_W, fc_b):
    del x2
    xp = jnp.transpose(x1.reshape(_N, _C, _T, _VM), (2, 0, 1, 3))
    xp = xp.reshape(_T, _N, _C * _VM)
    xp = jnp.pad(xp.astype(jnp.bfloat16),
                 ((0, 0), (0, 0), (0, _VMP - _C * _VM)))

    lhat = jnp.asarray(_LHAT)
    eye = jnp.eye(_V, dtype=jnp.float32)
    mflat = jnp.kron(eye, cheb_W0.T) + jnp.kron(lhat.T, cheb_W1.T)
    p0 = mflat @ W_ih_l0.T

    q = jnp.zeros((_M, _VMP, _G4), jnp.float32)
    v = np.arange(_V)
    for c in range(_C):
        for m in range(_M):
            q = q.at[m, c * _VM + v * _M + m, :].set(p0[v * _C + c, :])
    q = q.astype(jnp.bfloat16)

    c0m = (jnp.tile(cheb_b, _V) @ W_ih_l0.T + b_ih_l0 + b_hh_l0)[None, :]

    wf = jnp.concatenate([W_hh_l0.T, W_ih_l1.T],
                         axis=1).astype(jnp.bfloat16)
    w11 = W_hh_l1.T.astype(jnp.bfloat16)
    c1 = (b_ih_l1 + b_hh_l1)[None, :]

    return pl.pallas_call(
        _lstm_kernel,
        out_shape=jax.ShapeDtypeStruct((_B, fc_W.shape[0]), jnp.float32),
        scratch_shapes=[pltpu.VMEM((_M, _CH * _N, _G4), jnp.float32)],
    )(xp, q, c0m, wf, w11, c1, fc_W.T, fc_b[None, :])


# speedup vs baseline: 8.8701x; 1.0172x over previous
import numpy as np
import jax
import jax.numpy as jnp
from jax.experimental import pallas as pl
from jax.experimental.pallas import tpu as pltpu

_EDGE_SRC = np.array([0, 2, 4, 1, 3, 5, 6, 8, 10, 7, 9, 11, 13, 12, 2, 4, 14, 3, 5, 14, 8, 10, 13, 9, 11, 13, 12, 13], dtype=np.int64)
_EDGE_DST = np.array([2, 4, 14, 3, 5, 14, 8, 10, 13, 9, 11, 13, 14, 13, 0, 2, 4, 1, 3, 5, 6, 8, 10, 7, 9, 11, 13, 14], dtype=np.int64)
_V = 15


def _lhat() -> np.ndarray:
    deg = np.zeros((_V,), np.float64)
    for s in _EDGE_SRC:
        deg[s] += 1.0
    dinv = np.where(deg > 0, 1.0 / np.sqrt(np.maximum(deg, 1e-12)), 0.0)
    L = np.zeros((_V, _V), np.float64)
    for s, d in zip(_EDGE_SRC, _EDGE_DST):
        L[d, s] += -dinv[s] * dinv[d]
    return L.astype(np.float32)


_LHAT = _lhat()

_T, _B, _H, _C, _M = 300, 64, 128, 3, 2
_N = 32
_G4 = 4 * _H
_VM = _V * _M
_VMP = 128
_CH = 150
_NCH = _T // _CH
_UNPERM = np.array([(b % 2) * 32 + b // 2 for b in range(64)], dtype=np.int32)


def _gate(g, c_prev):
    i = jax.nn.sigmoid(g[:, :_H])
    f = jax.nn.sigmoid(g[:, _H:2 * _H])
    gg = jnp.tanh(g[:, 2 * _H:3 * _H])
    o = jax.nn.sigmoid(g[:, 3 * _H:])
    c = f * c_prev + i * gg
    h = o * jnp.tanh(c)
    return h, c


def _lstm_kernel(xp_ref, q_ref, c0m_ref, wf_ref, w11_ref, c1_ref, fcw_ref,
                 fcb_ref, out_ref, gcm_ref):
    z = jnp.zeros((_B, _H), jnp.float32)

    def substep(i_glob, j, st):
        h0, c0s, h1, c1s = st
        ga = jnp.dot(h0.astype(jnp.bfloat16), wf_ref[...],
                     preferred_element_type=jnp.float32)
        gb = jnp.dot(h1.astype(jnp.bfloat16), w11_ref[...],
                     preferred_element_type=jnp.float32)
        a0 = gcm_ref[0, pl.ds(j * _N, _N), :]
        a1 = gcm_ref[1, pl.ds(j * _N, _N), :]
        g0 = jnp.concatenate([a0, a1], axis=0) + ga[:, :_G4] + c0m_ref[...]
        g1 = ga[:, _G4:] + gb + c1_ref[...]
        h0n, c0n = _gate(g0, c0s)
        h1n, c1n = _gate(g1, c1s)
        first = i_glob == 0
        h1 = jnp.where(first, h1, h1n)
        c1s = jnp.where(first, c1s, c1n)
        return (h0n, c0n, h1, c1s)

    def chunk(ci, carry):
        base = ci * _CH
        xc = xp_ref[pl.ds(base, _CH), :, :].reshape(_CH * _N, _VMP)
        gcm_ref[0] = jnp.dot(xc, q_ref[0], preferred_element_type=jnp.float32)
        gcm_ref[1] = jnp.dot(xc, q_ref[1], preferred_element_type=jnp.float32)

        def step(jj, st):
            for u in range(25):
                st = substep(ci * _CH + 25 * jj + u, 25 * jj + u, st)
            return st

        return jax.lax.fori_loop(0, _CH // 25, step, carry)

    h0, c0s, h1, c1s = jax.lax.fori_loop(0, _NCH, chunk, (z, z, z, z))

    ga = jnp.dot(h0.astype(jnp.bfloat16), wf_ref[...],
                 preferred_element_type=jnp.float32)
    gb = jnp.dot(h1.astype(jnp.bfloat16), w11_ref[...],
                 preferred_element_type=jnp.float32)
    g1 = ga[:, _G4:] + gb + c1_ref[...]
    h1, _ = _gate(g1, c1s)

    logits = jnp.dot(h1, fcw_ref[...],
                     preferred_element_type=jnp.float32) + fcb_ref[...]
    m = jnp.max(logits, axis=1, keepdims=True)
    e = jnp.exp(logits - m)
    sm = e / jnp.sum(e, axis=1, keepdims=True)
    out_ref[...] = jnp.transpose(sm.reshape(_M, _N, -1), (1, 0, 2)).reshape(_B, -1)


def kernel(x1, x2, cheb_W0, cheb_W1, cheb_b, W_ih_l0, W_hh_l0, b_ih_l0,
           b_hh_l0, W_ih_l1, W_hh_l1, b_ih_l1, b_hh_l1, fc_W, fc_b):
    del x2
    xp = jnp.transpose(x1.reshape(_N, _C, _T, _VM), (2, 0, 1, 3))
    xp = xp.reshape(_T, _N, _C * _VM)
    xp = jnp.pad(xp.astype(jnp.bfloat16),
                 ((0, 0), (0, 0), (0, _VMP - _C * _VM)))

    lhat = jnp.asarray(_LHAT)
    eye = jnp.eye(_V, dtype=jnp.float32)
    mflat = jnp.kron(eye, cheb_W0.T) + jnp.kron(lhat.T, cheb_W1.T)
    p0 = mflat @ W_ih_l0.T

    q = jnp.zeros((_M, _VMP, _G4), jnp.float32)
    v = np.arange(_V)
    for c in range(_C):
        for m in range(_M):
            q = q.at[m, c * _VM + v * _M + m, :].set(p0[v * _C + c, :])
    q = q.astype(jnp.bfloat16)

    c0m = (jnp.tile(cheb_b, _V) @ W_ih_l0.T + b_ih_l0 + b_hh_l0)[None, :]

    wf = jnp.concatenate([W_hh_l0.T, W_ih_l1.T],
                         axis=1).astype(jnp.bfloat16)
    w11 = W_hh_l1.T.astype(jnp.bfloat16)
    c1 = (b_ih_l1 + b_hh_l1)[None, :]

    return pl.pallas_call(
        _lstm_kernel,
        out_shape=jax.ShapeDtypeStruct((_B, fc_W.shape[0]), jnp.float32),
        scratch_shapes=[pltpu.VMEM((_M, _CH * _N, _G4), jnp.float32)],
    )(xp, q, c0m, wf, w11, c1, fc_W.T, fc_b[None, :])


# speedup vs baseline: 8.9163x; 1.0052x over previous
import numpy as np
import jax
import jax.numpy as jnp
from jax.experimental import pallas as pl
from jax.experimental.pallas import tpu as pltpu

_EDGE_SRC = np.array([0, 2, 4, 1, 3, 5, 6, 8, 10, 7, 9, 11, 13, 12, 2, 4, 14, 3, 5, 14, 8, 10, 13, 9, 11, 13, 12, 13], dtype=np.int64)
_EDGE_DST = np.array([2, 4, 14, 3, 5, 14, 8, 10, 13, 9, 11, 13, 14, 13, 0, 2, 4, 1, 3, 5, 6, 8, 10, 7, 9, 11, 13, 14], dtype=np.int64)
_V = 15


def _lhat() -> np.ndarray:
    deg = np.zeros((_V,), np.float64)
    for s in _EDGE_SRC:
        deg[s] += 1.0
    dinv = np.where(deg > 0, 1.0 / np.sqrt(np.maximum(deg, 1e-12)), 0.0)
    L = np.zeros((_V, _V), np.float64)
    for s, d in zip(_EDGE_SRC, _EDGE_DST):
        L[d, s] += -dinv[s] * dinv[d]
    return L.astype(np.float32)


_LHAT = _lhat()

_T, _B, _H, _C, _M = 300, 64, 128, 3, 2
_N = 32
_G4 = 4 * _H
_VM = _V * _M
_VMP = 128
_CH = 150
_NCH = _T // _CH
_UNPERM = np.array([(b % 2) * 32 + b // 2 for b in range(64)], dtype=np.int32)


def _gate(g, c_prev):
    i = jax.nn.sigmoid(g[:, :_H])
    f = jax.nn.sigmoid(g[:, _H:2 * _H])
    gg = jnp.tanh(g[:, 2 * _H:3 * _H])
    o = jax.nn.sigmoid(g[:, 3 * _H:])
    c = f * c_prev + i * gg
    h = o * jnp.tanh(c)
    return h, c


def _lstm_kernel(xp_ref, q_ref, c0m_ref, wf_ref, w11_ref, c1_ref, fcw_ref,
                 fcb_ref, out_ref, gcm_ref):
    z = jnp.zeros((_B, _H), jnp.float32)

    def substep(i_glob, j, st):
        h0, c0s, h1, c1s = st
        ga = jnp.dot(h0.astype(jnp.bfloat16), wf_ref[...],
                     preferred_element_type=jnp.float32)
        gb = jnp.dot(h1.astype(jnp.bfloat16), w11_ref[...],
                     preferred_element_type=jnp.float32)
        a0 = gcm_ref[0, pl.ds(j * _N, _N), :]
        a1 = gcm_ref[1, pl.ds(j * _N, _N), :]
        g0 = jnp.concatenate([a0, a1], axis=0) + ga[:, :_G4]
        g1 = ga[:, _G4:] + gb + c1_ref[...]
        h0n, c0n = _gate(g0, c0s)
        h1n, c1n = _gate(g1, c1s)
        first = i_glob == 0
        h1 = jnp.where(first, h1, h1n)
        c1s = jnp.where(first, c1s, c1n)
        return (h0n, c0n, h1, c1s)

    def chunk(ci, carry):
        base = ci * _CH
        xc = xp_ref[pl.ds(base, _CH), :, :].reshape(_CH * _N, _VMP)
        c0m = c0m_ref[...]
        gcm_ref[0] = jnp.dot(xc, q_ref[0],
                             preferred_element_type=jnp.float32) + c0m
        gcm_ref[1] = jnp.dot(xc, q_ref[1],
                             preferred_element_type=jnp.float32) + c0m

        def step(jj, st):
            for u in range(25):
                st = substep(ci * _CH + 25 * jj + u, 25 * jj + u, st)
            return st

        return jax.lax.fori_loop(0, _CH // 25, step, carry)

    h0, c0s, h1, c1s = jax.lax.fori_loop(0, _NCH, chunk, (z, z, z, z))

    ga = jnp.dot(h0.astype(jnp.bfloat16), wf_ref[...],
                 preferred_element_type=jnp.float32)
    gb = jnp.dot(h1.astype(jnp.bfloat16), w11_ref[...],
                 preferred_element_type=jnp.float32)
    g1 = ga[:, _G4:] + gb + c1_ref[...]
    h1, _ = _gate(g1, c1s)

    logits = jnp.dot(h1, fcw_ref[...],
                     preferred_element_type=jnp.float32) + fcb_ref[...]
    m = jnp.max(logits, axis=1, keepdims=True)
    e = jnp.exp(logits - m)
    sm = e / jnp.sum(e, axis=1, keepdims=True)
    out_ref[...] = jnp.transpose(sm.reshape(_M, _N, -1), (1, 0, 2)).reshape(_B, -1)


def kernel(x1, x2, cheb_W0, cheb_W1, cheb_b, W_ih_l0, W_hh_l0, b_ih_l0,
           b_hh_l0, W_ih_l1, W_hh_l1, b_ih_l1, b_hh_l1, fc_W, fc_b):
    del x2
    xp = jnp.transpose(x1.reshape(_N, _C, _T, _VM), (2, 0, 1, 3))
    xp = xp.reshape(_T, _N, _C * _VM)
    xp = jnp.pad(xp.astype(jnp.bfloat16),
                 ((0, 0), (0, 0), (0, _VMP - _C * _VM)))

    lhat = jnp.asarray(_LHAT)
    eye = jnp.eye(_V, dtype=jnp.float32)
    mflat = jnp.kron(eye, cheb_W0.T) + jnp.kron(lhat.T, cheb_W1.T)
    p0 = mflat @ W_ih_l0.T

    q = jnp.zeros((_M, _VMP, _G4), jnp.float32)
    v = np.arange(_V)
    for c in range(_C):
        for m in range(_M):
            q = q.at[m, c * _VM + v * _M + m, :].set(p0[v * _C + c, :])
    q = q.astype(jnp.bfloat16)

    c0m = (jnp.tile(cheb_b, _V) @ W_ih_l0.T + b_ih_l0 + b_hh_l0)[None, :]

    wf = jnp.concatenate([W_hh_l0.T, W_ih_l1.T],
                         axis=1).astype(jnp.bfloat16)
    w11 = W_hh_l1.T.astype(jnp.bfloat16)
    c1 = (b_ih_l1 + b_hh_l1)[None, :]

    return pl.pallas_call(
        _lstm_kernel,
        out_shape=jax.ShapeDtypeStruct((_B, fc_W.shape[0]), jnp.float32),
        scratch_shapes=[pltpu.VMEM((_M, _CH * _N, _G4), jnp.float32)],
    )(xp, q, c0m, wf, w11, c1, fc_W.T, fc_b[None, :])
